# baseline (speedup 1.0000x reference)
import functools

import jax
import jax.numpy as jnp
from jax import lax
from jax.experimental import pallas as pl
from jax.experimental.pallas import tpu as pltpu
from jax.experimental.pallas import tpu_sc as plsc

_NC, _NS = 2, 16
_NW = _NC * _NS


def _tc_body(map_t, actor_t, map_pe, actor_pe, time_pe, pos,
             map_o, actor_o):
    S = map_t.shape[1]
    T = actor_t.shape[1]
    N = actor_t.shape[2]
    D = map_t.shape[-1]

    pos_all = pos[...]
    map_o[...] = map_t[...] + (map_pe[...] + pos_all[:S])[None]

    time_comb = (time_pe[:T] + pos_all[:T]).reshape(1, T, 1, D)
    actor_comb = (actor_pe[:N] + pos_all[:N]).reshape(1, 1, N, D)
    actor_o[...] = actor_t[...] + actor_comb + time_comb


def _tc_call(map_token, actor_token, map_pe_w, actor_pe_w, time_pe_w, pos_enc):
    B, S, D = map_token.shape
    _, T, N, _ = actor_token.shape
    whole = lambda shape: pl.BlockSpec(shape, lambda b: (0,) * len(shape))
    return pl.pallas_call(
        _tc_body,
        grid=(B,),
        in_specs=[
            pl.BlockSpec((1, S, D), lambda b: (b, 0, 0)),
            pl.BlockSpec((1, T, N, D), lambda b: (b, 0, 0, 0)),
            whole(map_pe_w.shape),
            whole(actor_pe_w.shape),
            whole(time_pe_w.shape),
            whole(pos_enc.shape),
        ],
        out_specs=[
            pl.BlockSpec((1, S, D), lambda b: (b, 0, 0)),
            pl.BlockSpec((1, T, N, D), lambda b: (b, 0, 0, 0)),
        ],
        out_shape=[
            jax.ShapeDtypeStruct((B, S, D), map_token.dtype),
            jax.ShapeDtypeStruct((B, T, N, D), actor_token.dtype),
        ],
    )(map_token, actor_token, map_pe_w, actor_pe_w, time_pe_w, pos_enc)


def _sc_light_body(B, T, L, D, light_hbm, lpe_hbm, tpe_hbm, pos_hbm,
                   out_hbm, pe_v, tm_v, tmp_v, buf_v):
    nvec = D // 16
    wid = lax.axis_index("s") * _NC + lax.axis_index("c")
    slabs_per_w = (B * T) // _NW

    pltpu.sync_copy(lpe_hbm.at[pl.ds(0, L)], pe_v)
    pltpu.sync_copy(pos_hbm.at[pl.ds(0, L)], tmp_v.at[pl.ds(0, L)])

    def comb_pe(n, _):
        for j in range(nvec):
            plsc.addupdate(pe_v.at[n, pl.ds(j * 16, 16)],
                           tmp_v[n, pl.ds(j * 16, 16)])
        return 0
    lax.fori_loop(0, L, comb_pe, 0)

    Tp = ((T + 7) // 8) * 8
    pltpu.sync_copy(tpe_hbm.at[pl.ds(0, Tp)], tm_v)
    pltpu.sync_copy(pos_hbm.at[pl.ds(0, Tp)], tmp_v)

    def comb_tm(n, _):
        for j in range(nvec):
            plsc.addupdate(tm_v.at[n, pl.ds(j * 16, 16)],
                           tmp_v[n, pl.ds(j * 16, 16)])
        return 0
    lax.fori_loop(0, T, comb_tm, 0)

    def slab(i, _):
        p = wid * slabs_per_w + i
        b = lax.rem(p, B)
        t = lax.div(p, B)
        pltpu.sync_copy(light_hbm.at[b, t], buf_v)
        tms = [tm_v[t, pl.ds(j * 16, 16)] for j in range(nvec)]

        def row(n, _):
            for j in range(nvec):
                plsc.addupdate(buf_v.at[n, pl.ds(j * 16, 16)],
                               pe_v[n, pl.ds(j * 16, 16)] + tms[j])
            return 0
        lax.fori_loop(0, L, row, 0)
        pltpu.sync_copy(buf_v, out_hbm.at[b, t])
        return 0
    lax.fori_loop(0, slabs_per_w, slab, 0)


def _sc_light_call(light_token, light_pe_w, time_pe_w, pos_enc):
    B, T, L, D = light_token.shape
    mesh = plsc.VectorSubcoreMesh(core_axis_name="c", subcore_axis_name="s",
                                  num_cores=_NC, num_subcores=_NS)
    fn = pl.kernel(
        functools.partial(_sc_light_body, B, T, L, D),
        out_type=jax.ShapeDtypeStruct((B, T, L, D), light_token.dtype),
        mesh=mesh,
        scratch_types=[
            pltpu.VMEM((L, D), jnp.float32),
            pltpu.VMEM(((T + 7) // 8 * 8, D), jnp.float32),
            pltpu.VMEM(((T + 7) // 8 * 8, D), jnp.float32),
            pltpu.VMEM((L, D), jnp.float32),
        ],
    )
    return fn(light_token, light_pe_w, time_pe_w, pos_enc)


def kernel(map_token, actor_token, light_token, map_pe_w, actor_pe_w,
           light_pe_w, time_pe_w, pos_enc):
    light_o = _sc_light_call(light_token, light_pe_w, time_pe_w, pos_enc)
    map_o, actor_o = _tc_call(map_token, actor_token, map_pe_w, actor_pe_w,
                              time_pe_w, pos_enc)
    return (map_o, actor_o, light_o)


# speedup vs baseline: 1.0392x; 1.0392x over previous
import functools

import jax
import jax.numpy as jnp
from jax import lax
from jax.experimental import pallas as pl
from jax.experimental.pallas import tpu as pltpu
from jax.experimental.pallas import tpu_sc as plsc

_NC, _NS = 2, 16
_NW = _NC * _NS


def _tc_body(actor_t, actor_pe, time_pe, pos, actor_o):
    T = actor_t.shape[1]
    N = actor_t.shape[2]
    D = actor_t.shape[-1]
    pos_all = pos[...]
    time_comb = (time_pe[:T] + pos_all[:T]).reshape(1, T, 1, D)
    actor_comb = (actor_pe[:N] + pos_all[:N]).reshape(1, 1, N, D)
    actor_o[...] = actor_t[...] + actor_comb + time_comb


def _tc_call(actor_token, actor_pe_w, time_pe_w, pos_enc):
    B, T, N, D = actor_token.shape
    whole = lambda shape: pl.BlockSpec(shape, lambda b: (0,) * len(shape))
    return pl.pallas_call(
        _tc_body,
        grid=(B,),
        in_specs=[
            pl.BlockSpec((1, T, N, D), lambda b: (b, 0, 0, 0)),
            whole(actor_pe_w.shape),
            whole(time_pe_w.shape),
            whole(pos_enc.shape),
        ],
        out_specs=pl.BlockSpec((1, T, N, D), lambda b: (b, 0, 0, 0)),
        out_shape=jax.ShapeDtypeStruct((B, T, N, D), actor_token.dtype),
    )(actor_token, actor_pe_w, time_pe_w, pos_enc)


_MROWS = 80
_LTT = 2


def _sc_body(B, S, T, L, D,
             map_hbm, light_hbm, mpe_hbm, lpe_hbm, tpe_hbm, pos_hbm,
             map_out, light_out,
             mi0, mi1, mo0, mo1, pet, ptmp,
             li0, li1, lo0, lo1, lpe_v, tm_v,
             si0, si1, so0, so1):
    nvec = D // 16
    wid = lax.axis_index("s") * _NC + lax.axis_index("c")

    Tp = tm_v.shape[0]
    pltpu.sync_copy(lpe_hbm.at[pl.ds(0, L)], lpe_v)
    pltpu.sync_copy(pos_hbm.at[pl.ds(0, L)], ptmp.at[pl.ds(0, L)])

    def comb_lpe(n, _):
        for j in range(nvec):
            plsc.addupdate(lpe_v.at[n, pl.ds(j * 16, 16)],
                           ptmp[n, pl.ds(j * 16, 16)])
        return 0
    lax.fori_loop(0, L, comb_lpe, 0)

    pltpu.sync_copy(tpe_hbm.at[pl.ds(0, Tp)], tm_v)
    pltpu.sync_copy(pos_hbm.at[pl.ds(0, Tp)], ptmp.at[pl.ds(0, Tp)])

    def comb_tm(n, _):
        for j in range(nvec):
            plsc.addupdate(tm_v.at[n, pl.ds(j * 16, 16)],
                           ptmp[n, pl.ds(j * 16, 16)])
        return 0
    lax.fori_loop(0, T, comb_tm, 0)

    n_mslab = (S // _MROWS) * B
    iters_m = (n_mslab + _NW - 1) // _NW
    ibufs, obufs = (mi0, mi1), (mo0, mo1)
    isems, osems = (si0, si1), (so0, so1)

    def m_p(i):
        return wid * iters_m + i

    def m_issue_in(i):
        p = m_p(i)

        @pl.when(p < n_mslab)
        def _():
            c = lax.div(p, B)
            b = lax.rem(p, B)
            pltpu.async_copy(
                map_hbm.at[b, pl.ds(c * _MROWS, _MROWS)],
                ibufs[i % 2], isems[i % 2])

    cprev = jnp.int32(-1)
    pend_out = []
    for i in range(min(2, iters_m)):
        m_issue_in(i)
    for i in range(iters_m):
        sl = i % 2
        p = m_p(i)
        valid = p < n_mslab
        c = lax.div(p, B)
        b = lax.rem(p, B)

        if i >= 2:
            @pl.when(m_p(i - 2) < n_mslab)
            def _(sl=sl):
                pltpu.make_async_copy(
                    obufs[sl], map_hbm.at[0, pl.ds(0, _MROWS)],
                    osems[sl]).wait()

        @pl.when(valid)
        def _(sl=sl, p=p, c=c, b=b, cprev=cprev):
            @pl.when(c != cprev)
            def _():
                pltpu.sync_copy(mpe_hbm.at[pl.ds(c * _MROWS, _MROWS)], pet)
                pltpu.sync_copy(pos_hbm.at[pl.ds(c * _MROWS, _MROWS)], ptmp)

                def comb(n, _):
                    for j in range(nvec):
                        plsc.addupdate(pet.at[n, pl.ds(j * 16, 16)],
                                       ptmp[n, pl.ds(j * 16, 16)])
                    return 0
                lax.fori_loop(0, _MROWS, comb, 0)

            pltpu.make_async_copy(
                map_hbm.at[b, pl.ds(c * _MROWS, _MROWS)],
                ibufs[sl], isems[sl]).wait()

            def row(n, _):
                for j in range(nvec):
                    d = pl.ds(j * 16, 16)
                    obufs[sl][n, d] = ibufs[sl][n, d] + pet[n, d]
                return 0
            lax.fori_loop(0, _MROWS, row, 0)
            pltpu.async_copy(
                obufs[sl], map_out.at[b, pl.ds(c * _MROWS, _MROWS)],
                osems[sl])
        cprev = jnp.where(valid, c, cprev)
        pend_out.append(i)
        if len(pend_out) > 2:
            pend_out.pop(0)
        if i + 2 < iters_m:
            m_issue_in(i + 2)
    for i in pend_out:
        sl = i % 2
        p = m_p(i)

        @pl.when(p < n_mslab)
        def _(sl=sl):
            pltpu.make_async_copy(
                obufs[sl], map_hbm.at[0, pl.ds(0, _MROWS)],
                osems[sl]).wait()

    ntc = T // _LTT
    n_lslab = B * ntc
    iters_l = (n_lslab + _NW - 1) // _NW
    libufs, lobufs = (li0, li1), (lo0, lo1)

    def l_p(i):
        return wid * iters_l + i

    def l_issue_in(i):
        p = l_p(i)

        @pl.when(p < n_lslab)
        def _():
            b = lax.div(p, ntc)
            t0 = lax.rem(p, ntc) * _LTT
            pltpu.async_copy(
                light_hbm.at[b, pl.ds(t0, _LTT)],
                libufs[i % 2], isems[i % 2])

    lpend = []
    for i in range(min(2, iters_l)):
        l_issue_in(i)
    for i in range(iters_l):
        sl = i % 2
        p = l_p(i)
        b = lax.div(p, ntc)
        t0 = lax.rem(p, ntc) * _LTT

        if i >= 2:
            @pl.when(l_p(i - 2) < n_lslab)
            def _(sl=sl):
                pltpu.make_async_copy(
                    lobufs[sl], light_hbm.at[0, pl.ds(0, _LTT)],
                    osems[sl]).wait()

        @pl.when(p < n_lslab)
        def _(sl=sl, p=p, b=b, t0=t0):
            pltpu.make_async_copy(
                light_hbm.at[b, pl.ds(t0, _LTT)],
                libufs[sl], isems[sl]).wait()
            for tt in range(_LTT):
                t = t0 + tt
                tms = [tm_v[t, pl.ds(j * 16, 16)] for j in range(nvec)]

                def row(n, _, tt=tt, tms=tms):
                    for j in range(nvec):
                        d = pl.ds(j * 16, 16)
                        lobufs[sl][tt, n, d] = (libufs[sl][tt, n, d]
                                                + lpe_v[n, d] + tms[j])
                    return 0
                lax.fori_loop(0, L, row, 0)
            pltpu.async_copy(
                lobufs[sl], light_out.at[b, pl.ds(t0, _LTT)],
                osems[sl])
        lpend.append(i)
        if len(lpend) > 2:
            lpend.pop(0)
        if i + 2 < iters_l:
            l_issue_in(i + 2)
    for i in lpend:
        sl = i % 2
        p = l_p(i)

        @pl.when(p < n_lslab)
        def _(sl=sl):
            pltpu.make_async_copy(
                lobufs[sl], light_hbm.at[0, pl.ds(0, _LTT)],
                osems[sl]).wait()


def _sc_call(map_token, light_token, map_pe_w, light_pe_w, time_pe_w, pos_enc):
    B, S, D = map_token.shape
    _, T, L, _ = light_token.shape
    Tp = (T + 7) // 8 * 8
    mesh = plsc.VectorSubcoreMesh(core_axis_name="c", subcore_axis_name="s",
                                  num_cores=_NC, num_subcores=_NS)
    f32 = jnp.float32
    fn = pl.kernel(
        functools.partial(_sc_body, B, S, T, L, D),
        out_type=(jax.ShapeDtypeStruct((B, S, D), map_token.dtype),
                  jax.ShapeDtypeStruct((B, T, L, D), light_token.dtype)),
        mesh=mesh,
        scratch_types=[
            pltpu.VMEM((_MROWS, D), f32),
            pltpu.VMEM((_MROWS, D), f32),
            pltpu.VMEM((_MROWS, D), f32),
            pltpu.VMEM((_MROWS, D), f32),
            pltpu.VMEM((_MROWS, D), f32),
            pltpu.VMEM((_MROWS, D), f32),
            pltpu.VMEM((_LTT, L, D), f32),
            pltpu.VMEM((_LTT, L, D), f32),
            pltpu.VMEM((_LTT, L, D), f32),
            pltpu.VMEM((_LTT, L, D), f32),
            pltpu.VMEM((L, D), f32),
            pltpu.VMEM((Tp, D), f32),
            pltpu.SemaphoreType.DMA,
            pltpu.SemaphoreType.DMA,
            pltpu.SemaphoreType.DMA,
            pltpu.SemaphoreType.DMA,
        ],
    )
    return fn(map_token, light_token, map_pe_w, light_pe_w, time_pe_w, pos_enc)


def kernel(map_token, actor_token, light_token, map_pe_w, actor_pe_w,
           light_pe_w, time_pe_w, pos_enc):
    map_o, light_o = _sc_call(map_token, light_token, map_pe_w, light_pe_w,
                              time_pe_w, pos_enc)
    actor_o = _tc_call(actor_token, actor_pe_w, time_pe_w, pos_enc)
    return (map_o, actor_o, light_o)


# speedup vs baseline: 1.1094x; 1.0676x over previous
import functools

import jax
import jax.numpy as jnp
from jax import lax
from jax.experimental import pallas as pl
from jax.experimental.pallas import tpu as pltpu
from jax.experimental.pallas import tpu_sc as plsc

_NC, _NS = 2, 16
_NW = _NC * _NS


def _tc_body(actor_t, actor_pe, time_pe, pos, actor_o):
    T = actor_t.shape[1]
    N = actor_t.shape[2]
    D = actor_t.shape[-1]
    pos_all = pos[...]
    time_comb = (time_pe[:T] + pos_all[:T]).reshape(1, T, 1, D)
    actor_comb = (actor_pe[:N] + pos_all[:N]).reshape(1, 1, N, D)
    actor_o[...] = actor_t[...] + actor_comb + time_comb


def _tc_call(actor_token, actor_pe_w, time_pe_w, pos_enc):
    B, T, N, D = actor_token.shape
    whole = lambda shape: pl.BlockSpec(shape, lambda b: (0,) * len(shape))
    return pl.pallas_call(
        _tc_body,
        grid=(B,),
        in_specs=[
            pl.BlockSpec((1, T, N, D), lambda b: (b, 0, 0, 0)),
            whole(actor_pe_w.shape),
            whole(time_pe_w.shape),
            whole(pos_enc.shape),
        ],
        out_specs=pl.BlockSpec((1, T, N, D), lambda b: (b, 0, 0, 0)),
        out_shape=jax.ShapeDtypeStruct((B, T, N, D), actor_token.dtype),
    )(actor_token, actor_pe_w, time_pe_w, pos_enc)


_MROWS = 80
_LTT = 2
_NBUF = 5
_DEPTH = 3


def _sc_body(B, S, T, L, D,
             map_hbm, light_hbm, mpe_hbm, lpe_hbm, tpe_hbm, pos_hbm,
             map_out, light_out,
             m0, m1, m2, m3, m4, pet, ptmp,
             l0, l1, l2, l3, l4, lpe_v, tm_v,
             si0, si1, si2, si3, si4, so0, so1, so2, so3, so4):
    nvec = D // 16
    wid = lax.axis_index("s") * _NC + lax.axis_index("c")
    mbufs = (m0, m1, m2, m3, m4)
    lbufs = (l0, l1, l2, l3, l4)
    isems = (si0, si1, si2, si3, si4)
    osems = (so0, so1, so2, so3, so4)

    Tp = tm_v.shape[0]
    pltpu.sync_copy(lpe_hbm.at[pl.ds(0, L)], lpe_v)
    pltpu.sync_copy(pos_hbm.at[pl.ds(0, L)], ptmp.at[pl.ds(0, L)])

    def comb_lpe(n, _):
        for j in range(nvec):
            plsc.addupdate(lpe_v.at[n, pl.ds(j * 16, 16)],
                           ptmp[n, pl.ds(j * 16, 16)])
        return 0
    lax.fori_loop(0, L, comb_lpe, 0)

    pltpu.sync_copy(tpe_hbm.at[pl.ds(0, Tp)], tm_v)
    pltpu.sync_copy(pos_hbm.at[pl.ds(0, Tp)], ptmp.at[pl.ds(0, Tp)])

    def comb_tm(n, _):
        for j in range(nvec):
            plsc.addupdate(tm_v.at[n, pl.ds(j * 16, 16)],
                           ptmp[n, pl.ds(j * 16, 16)])
        return 0
    lax.fori_loop(0, T, comb_tm, 0)

    n_mslab = (S // _MROWS) * B
    iters_m = (n_mslab + _NW - 1) // _NW

    def m_p(i):
        return wid * iters_m + i

    def m_src(i):
        p = m_p(i)
        c = lax.div(p, B)
        b = lax.rem(p, B)
        return map_hbm.at[b, pl.ds(c * _MROWS, _MROWS)]

    def m_issue_in(i):
        @pl.when(m_p(i) < n_mslab)
        def _():
            pltpu.async_copy(m_src(i), mbufs[i % _NBUF], isems[i % _NBUF])

    for i in range(min(_DEPTH, iters_m)):
        m_issue_in(i)
    cprev = jnp.int32(-1)
    for i in range(iters_m):
        sl = i % _NBUF
        p = m_p(i)
        c = lax.div(p, B)
        b = lax.rem(p, B)

        @pl.when(p < n_mslab)
        def _(sl=sl, i=i, c=c, b=b, cprev=cprev):
            @pl.when(c != cprev)
            def _():
                pltpu.sync_copy(mpe_hbm.at[pl.ds(c * _MROWS, _MROWS)], pet)
                pltpu.sync_copy(pos_hbm.at[pl.ds(c * _MROWS, _MROWS)], ptmp)

                def comb(n, _):
                    for j in range(nvec):
                        plsc.addupdate(pet.at[n, pl.ds(j * 16, 16)],
                                       ptmp[n, pl.ds(j * 16, 16)])
                    return 0
                lax.fori_loop(0, _MROWS, comb, 0)

            pltpu.make_async_copy(m_src(i), mbufs[sl], isems[sl]).wait()

            def row(n, _):
                for j in range(nvec):
                    d = pl.ds(j * 16, 16)
                    plsc.addupdate(mbufs[sl].at[n, d], pet[n, d])
                return 0
            lax.fori_loop(0, _MROWS, row, 0)
            pltpu.async_copy(
                mbufs[sl], map_out.at[b, pl.ds(c * _MROWS, _MROWS)],
                osems[sl])
        cprev = jnp.where(p < n_mslab, c, cprev)

        jj = i + _DEPTH
        if jj < iters_m:
            sj = jj % _NBUF
            prev = jj - _NBUF
            if prev >= 0:
                @pl.when(m_p(prev) < n_mslab)
                def _(sj=sj):
                    pltpu.make_async_copy(
                        mbufs[sj], map_hbm.at[0, pl.ds(0, _MROWS)],
                        osems[sj]).wait()
            m_issue_in(jj)
    for i in range(max(0, iters_m - _NBUF), iters_m):
        @pl.when(m_p(i) < n_mslab)
        def _(sl=i % _NBUF):
            pltpu.make_async_copy(
                mbufs[sl], map_hbm.at[0, pl.ds(0, _MROWS)],
                osems[sl]).wait()

    ntc = T // _LTT
    n_lslab = B * ntc
    iters_l = (n_lslab + _NW - 1) // _NW

    def l_p(i):
        return wid * iters_l + i

    def l_src(i):
        p = l_p(i)
        b = lax.div(p, ntc)
        t0 = lax.rem(p, ntc) * _LTT
        return light_hbm.at[b, pl.ds(t0, _LTT)]

    def l_issue_in(i):
        @pl.when(l_p(i) < n_lslab)
        def _():
            pltpu.async_copy(l_src(i), lbufs[i % _NBUF], isems[i % _NBUF])

    for i in range(min(_DEPTH, iters_l)):
        l_issue_in(i)
    for i in range(iters_l):
        sl = i % _NBUF
        p = l_p(i)
        t0 = lax.rem(p, ntc) * _LTT
        b = lax.div(p, ntc)

        @pl.when(p < n_lslab)
        def _(sl=sl, i=i, b=b, t0=t0):
            pltpu.make_async_copy(l_src(i), lbufs[sl], isems[sl]).wait()
            for tt in range(_LTT):
                t = t0 + tt
                tms = [tm_v[t, pl.ds(j * 16, 16)] for j in range(nvec)]

                def row(n, _, tt=tt, tms=tms):
                    for j in range(nvec):
                        d = pl.ds(j * 16, 16)
                        plsc.addupdate(lbufs[sl].at[tt, n, d],
                                       lpe_v[n, d] + tms[j])
                    return 0
                lax.fori_loop(0, L, row, 0)
            pltpu.async_copy(
                lbufs[sl], light_out.at[b, pl.ds(t0, _LTT)],
                osems[sl])

        jj = i + _DEPTH
        if jj < iters_l:
            sj = jj % _NBUF
            prev = jj - _NBUF
            if prev >= 0:
                @pl.when(l_p(prev) < n_lslab)
                def _(sj=sj):
                    pltpu.make_async_copy(
                        lbufs[sj], light_hbm.at[0, pl.ds(0, _LTT)],
                        osems[sj]).wait()
            l_issue_in(jj)
    for i in range(max(0, iters_l - _NBUF), iters_l):
        @pl.when(l_p(i) < n_lslab)
        def _(sl=i % _NBUF):
            pltpu.make_async_copy(
                lbufs[sl], light_hbm.at[0, pl.ds(0, _LTT)],
                osems[sl]).wait()


def _sc_call(map_token, light_token, map_pe_w, light_pe_w, time_pe_w, pos_enc):
    B, S, D = map_token.shape
    _, T, L, _ = light_token.shape
    Tp = (T + 7) // 8 * 8
    mesh = plsc.VectorSubcoreMesh(core_axis_name="c", subcore_axis_name="s",
                                  num_cores=_NC, num_subcores=_NS)
    f32 = jnp.float32
    fn = pl.kernel(
        functools.partial(_sc_body, B, S, T, L, D),
        out_type=(jax.ShapeDtypeStruct((B, S, D), map_token.dtype),
                  jax.ShapeDtypeStruct((B, T, L, D), light_token.dtype)),
        mesh=mesh,
        scratch_types=(
            [pltpu.VMEM((_MROWS, D), f32) for _ in range(_NBUF)]
            + [pltpu.VMEM((_MROWS, D), f32),
               pltpu.VMEM((_MROWS, D), f32)]
            + [pltpu.VMEM((_LTT, L, D), f32) for _ in range(_NBUF)]
            + [pltpu.VMEM((L, D), f32),
               pltpu.VMEM((Tp, D), f32)]
            + [pltpu.SemaphoreType.DMA for _ in range(2 * _NBUF)]
        ),
    )
    return fn(map_token, light_token, map_pe_w, light_pe_w, time_pe_w, pos_enc)


def kernel(map_token, actor_token, light_token, map_pe_w, actor_pe_w,
           light_pe_w, time_pe_w, pos_enc):
    map_o, light_o = _sc_call(map_token, light_token, map_pe_w, light_pe_w,
                              time_pe_w, pos_enc)
    actor_o = _tc_call(actor_token, actor_pe_w, time_pe_w, pos_enc)
    return (map_o, actor_o, light_o)


# speedup vs baseline: 1.6445x; 1.4823x over previous
import jax
import jax.numpy as jnp
from jax.experimental import pallas as pl


def _qpe_body(map_t, actor_t, light_t, map_pe, actor_pe, light_pe, time_pe,
              pos, map_o, actor_o, light_o):
    S = map_t.shape[1]
    T = actor_t.shape[1]
    N = actor_t.shape[2]
    L = light_t.shape[2]
    D = map_t.shape[-1]
    pos_all = pos[...]
    map_o[...] = map_t[...] + (map_pe[...] + pos_all[:S])[None]
    time_comb = (time_pe[:T] + pos_all[:T]).reshape(1, T, 1, D)
    actor_comb = (actor_pe[:N] + pos_all[:N]).reshape(1, 1, N, D)
    actor_o[...] = actor_t[...] + actor_comb + time_comb
    light_comb = (light_pe[:L] + pos_all[:L]).reshape(1, 1, L, D)
    light_o[...] = light_t[...] + light_comb + time_comb


def kernel(map_token, actor_token, light_token, map_pe_w, actor_pe_w,
           light_pe_w, time_pe_w, pos_enc):
    B, S, D = map_token.shape
    _, T, N, _ = actor_token.shape
    L = light_token.shape[2]
    whole = lambda shape: pl.BlockSpec(shape, lambda b: (0,) * len(shape))
    outs = pl.pallas_call(
        _qpe_body,
        grid=(B,),
        in_specs=[
            pl.BlockSpec((1, S, D), lambda b: (b, 0, 0)),
            pl.BlockSpec((1, T, N, D), lambda b: (b, 0, 0, 0)),
            pl.BlockSpec((1, T, L, D), lambda b: (b, 0, 0, 0)),
            whole(map_pe_w.shape),
            whole(actor_pe_w.shape),
            whole(light_pe_w.shape),
            whole(time_pe_w.shape),
            whole(pos_enc.shape),
        ],
        out_specs=[
            pl.BlockSpec((1, S, D), lambda b: (b, 0, 0)),
            pl.BlockSpec((1, T, N, D), lambda b: (b, 0, 0, 0)),
            pl.BlockSpec((1, T, L, D), lambda b: (b, 0, 0, 0)),
        ],
        out_shape=[
            jax.ShapeDtypeStruct((B, S, D), map_token.dtype),
            jax.ShapeDtypeStruct((B, T, N, D), actor_token.dtype),
            jax.ShapeDtypeStruct((B, T, L, D), light_token.dtype),
        ],
    )(map_token, actor_token, light_token, map_pe_w, actor_pe_w,
      light_pe_w, time_pe_w, pos_enc)
    return tuple(outs)
